# baseline (speedup 1.0000x reference)
import functools

import jax
import jax.numpy as jnp
from jax import lax
from jax.experimental import pallas as pl
from jax.experimental.pallas import tpu as pltpu
from jax.experimental.pallas import tpu_sc as plsc

N = 10000
E = 320000
D = 128
G = 64
EPS = 1e-5

NC = 2
NS = 16
NW = NC * NS
EPW = E // NW
K = 128
CH = 80
HCH = CH // 2
E_PAD = NW * CH * K
NP = 10240
RPS = NP // NS


@functools.cache
def _sc_kernels():
    mesh = plsc.VectorSubcoreMesh(core_axis_name="c", subcore_axis_name="s",
                                  num_cores=NC, num_subcores=NS)

    @functools.partial(
        pl.kernel,
        out_type=jax.ShapeDtypeStruct((NC, NP, D), jnp.float32),
        mesh=mesh,
        scratch_types=[
            pltpu.VMEM((CH, K), jnp.int32),
            pltpu.VMEM((K, D), jnp.float32),
            pltpu.VMEM_SHARED((NP, D), jnp.float32),
        ],
    )
    def sc_deg(dst_hbm, ones_hbm, zagg_hbm, out_hbm, dst_v, ones_v, acc):
        c = lax.axis_index("c")
        s = lax.axis_index("s")
        wid = c * NS + s
        pltpu.sync_copy(dst_hbm.at[wid], dst_v)
        pltpu.sync_copy(ones_hbm, ones_v)
        pltpu.sync_copy(zagg_hbm, acc.at[pl.ds(s * RPS, RPS)])
        plsc.subcore_barrier()

        def step(j, carry):
            pltpu.sync_copy(ones_v, acc.at[dst_v.at[j]], add=True)
            return carry

        lax.fori_loop(0, CH, step, 0)
        plsc.subcore_barrier()
        pltpu.sync_copy(acc.at[pl.ds(s * RPS, RPS)],
                        out_hbm.at[c, pl.ds(s * RPS, RPS)])

    @functools.partial(
        pl.kernel,
        out_type=jax.ShapeDtypeStruct((NC, NP, D), jnp.float32),
        mesh=mesh,
        scratch_types=[
            pltpu.VMEM((HCH, K), jnp.int32),
            pltpu.VMEM((HCH, K), jnp.int32),
            pltpu.VMEM((K, D), jnp.float32),
            pltpu.VMEM((K, D), jnp.float32),
            pltpu.SemaphoreType.DMA,
            pltpu.SemaphoreType.DMA,
            pltpu.VMEM_SHARED((NP, D), jnp.float32),
        ],
    )
    def sc_agg(hs_hbm, src_hbm, dst_hbm, zagg_hbm, out_hbm,
               src_v, dst_v, rows0, rows1, sem0, sem1, acc):
        c = lax.axis_index("c")
        s = lax.axis_index("s")
        wid = c * NS + s
        pltpu.sync_copy(zagg_hbm, acc.at[pl.ds(s * RPS, RPS)])
        plsc.subcore_barrier()

        def run_half(h):
            pltpu.sync_copy(src_hbm.at[wid, pl.ds(h * HCH, HCH)], src_v)
            pltpu.sync_copy(dst_hbm.at[wid, pl.ds(h * HCH, HCH)], dst_v)
            pltpu.async_copy(hs_hbm.at[src_v.at[0]], rows0, sem0)

            def step(t, carry):
                j = 2 * t
                pltpu.make_async_copy(hs_hbm.at[src_v.at[j]], rows0,
                                      sem0).wait()
                pltpu.async_copy(hs_hbm.at[src_v.at[j + 1]], rows1, sem1)
                pltpu.sync_copy(rows0, acc.at[dst_v.at[j]], add=True)
                pltpu.make_async_copy(hs_hbm.at[src_v.at[j + 1]], rows1,
                                      sem1).wait()
                pltpu.async_copy(hs_hbm.at[src_v.at[j + 2]], rows0, sem0)
                pltpu.sync_copy(rows1, acc.at[dst_v.at[j + 1]], add=True)
                return carry

            lax.fori_loop(0, HCH // 2 - 1, step, 0)
            jt = HCH - 2
            pltpu.make_async_copy(hs_hbm.at[src_v.at[jt]], rows0, sem0).wait()
            pltpu.async_copy(hs_hbm.at[src_v.at[HCH - 1]], rows1, sem1)
            pltpu.sync_copy(rows0, acc.at[dst_v.at[jt]], add=True)
            pltpu.make_async_copy(hs_hbm.at[src_v.at[HCH - 1]], rows1,
                                  sem1).wait()
            pltpu.sync_copy(rows1, acc.at[dst_v.at[HCH - 1]], add=True)

        run_half(0)
        run_half(1)

        plsc.subcore_barrier()
        pltpu.sync_copy(acc.at[pl.ds(s * RPS, RPS)],
                        out_hbm.at[c, pl.ds(s * RPS, RPS)])

    return sc_deg, sc_agg


def _dis_from_deg(deg):
    d = deg[0, :N, :1] + deg[1, :N, :1] + 1.0
    return lax.rsqrt(jnp.maximum(d, 1.0))


def _tc_dense1(x_ref, w_ref, deg_ref, out_ref):
    dis = _dis_from_deg(deg_ref[...])
    h = jnp.dot(x_ref[...], w_ref[...], preferred_element_type=jnp.float32)
    out_ref[...] = h * dis


def _bn_relu(agg, hs, dis, b, gamma, beta):
    h = dis * (agg[0, :N] + agg[1, :N] + hs) + b
    mean = jnp.mean(h, axis=0, keepdims=True)
    var = jnp.mean((h - mean) ** 2, axis=0, keepdims=True)
    h = (h - mean) * lax.rsqrt(var + EPS) * gamma + beta
    return jnp.maximum(h, 0.0)


def _tc_dense2(agg_ref, hs_ref, deg_ref, b_ref, g_ref, be_ref, w_ref, out_ref):
    dis = _dis_from_deg(deg_ref[...])
    h = _bn_relu(agg_ref[...], hs_ref[...], dis, b_ref[...], g_ref[...],
                 be_ref[...])
    h2 = jnp.dot(h, w_ref[...], preferred_element_type=jnp.float32)
    out_ref[...] = h2 * dis


def _tc_dense3(agg_ref, hs_ref, deg_ref, b_ref, g_ref, be_ref, batch_ref,
               out_ref):
    dis = _dis_from_deg(deg_ref[...])
    h = _bn_relu(agg_ref[...], hs_ref[...], dis, b_ref[...], g_ref[...],
                 be_ref[...])
    seg = lax.broadcasted_iota(jnp.int32, (N, G), 1)
    onehot = (batch_ref[...] == seg).astype(jnp.float32)
    sums = lax.dot_general(onehot, h, (((0,), (0,)), ((), ())),
                           preferred_element_type=jnp.float32)
    counts = jnp.sum(onehot, axis=0)[:, None]
    out_ref[...] = sums / jnp.maximum(counts, 1.0)


def kernel(x, edge_index, batch, W1, b1, gamma1, beta1, W2, b2, gamma2, beta2):
    pad_src = jnp.zeros((E_PAD - E,), jnp.int32)
    pad_dst = N + (jnp.arange(E_PAD - E, dtype=jnp.int32) % (NP - N))
    src = jnp.concatenate([edge_index[0], pad_src]).reshape(NW, CH, K)
    dst = jnp.concatenate([edge_index[1], pad_dst]).reshape(NW, CH, K)
    batch2 = batch.reshape(N, 1)
    b1r, g1r, be1r = b1.reshape(1, D), gamma1.reshape(1, D), beta1.reshape(1, D)
    b2r, g2r, be2r = b2.reshape(1, D), gamma2.reshape(1, D), beta2.reshape(1, D)

    ones_deg = jnp.ones((K, D), jnp.float32)
    zagg = jnp.zeros((RPS, D), jnp.float32)

    sc_deg, sc_agg = _sc_kernels()
    deg = sc_deg(dst, ones_deg, zagg)

    hs1 = pl.pallas_call(
        _tc_dense1,
        out_shape=jax.ShapeDtypeStruct((N, D), jnp.float32),
    )(x, W1, deg)

    agg1 = sc_agg(hs1, src, dst, zagg)

    hs2 = pl.pallas_call(
        _tc_dense2,
        out_shape=jax.ShapeDtypeStruct((N, D), jnp.float32),
    )(agg1, hs1, deg, b1r, g1r, be1r, W2)

    agg2 = sc_agg(hs2, src, dst, zagg)

    out = pl.pallas_call(
        _tc_dense3,
        out_shape=jax.ShapeDtypeStruct((G, D), jnp.float32),
    )(agg2, hs2, deg, b2r, g2r, be2r, batch2)

    return out


# speedup vs baseline: 2.8486x; 2.8486x over previous
import functools

import jax
import jax.numpy as jnp
from jax import lax
from jax.experimental import pallas as pl
from jax.experimental.pallas import tpu as pltpu
from jax.experimental.pallas import tpu_sc as plsc

N = 10000
E = 320000
D = 128
G = 64
EPS = 1e-5

NC = 2
NS = 16
NW = NC * NS
EPW = E // NW
K = 128
CH = 80
HCH = CH // 2
E_PAD = NW * CH * K
NP = 10240
RPS = NP // NS


@functools.cache
def _sc_kernels():
    mesh = plsc.VectorSubcoreMesh(core_axis_name="c", subcore_axis_name="s",
                                  num_cores=NC, num_subcores=NS)

    @functools.partial(
        pl.kernel,
        out_type=jax.ShapeDtypeStruct((NC, NP, D), jnp.float32),
        mesh=mesh,
        scratch_types=[
            pltpu.VMEM((CH, K), jnp.int32),
            pltpu.VMEM((K, D), jnp.float32),
            pltpu.VMEM_SHARED((NP, D), jnp.float32),
        ],
    )
    def sc_deg(dst_hbm, ones_hbm, zagg_hbm, out_hbm, dst_v, ones_v, acc):
        c = lax.axis_index("c")
        s = lax.axis_index("s")
        wid = c * NS + s
        pltpu.sync_copy(dst_hbm.at[wid], dst_v)
        pltpu.sync_copy(ones_hbm, ones_v)
        pltpu.sync_copy(zagg_hbm, acc.at[pl.ds(s * RPS, RPS)])
        plsc.subcore_barrier()

        def step(j, carry):
            pltpu.sync_copy(ones_v, acc.at[dst_v.at[j]], add=True)
            return carry

        lax.fori_loop(0, CH, step, 0)
        plsc.subcore_barrier()
        pltpu.sync_copy(acc.at[pl.ds(s * RPS, RPS)],
                        out_hbm.at[c, pl.ds(s * RPS, RPS)])

    @functools.partial(
        pl.kernel,
        out_type=jax.ShapeDtypeStruct((NC, NP, D), jnp.float32),
        mesh=mesh,
        scratch_types=[
            pltpu.VMEM((HCH, K), jnp.int32),
            pltpu.VMEM((HCH, K), jnp.int32),
            pltpu.VMEM((K, D), jnp.float32),
            pltpu.VMEM((K, D), jnp.float32),
            pltpu.SemaphoreType.DMA,
            pltpu.SemaphoreType.DMA,
            pltpu.VMEM_SHARED((NP, D), jnp.float32),
        ],
    )
    def sc_agg(hs_hbm, src_hbm, dst_hbm, zagg_hbm, out_hbm,
               src_v, dst_v, rows0, rows1, sem0, sem1, acc):
        c = lax.axis_index("c")
        s = lax.axis_index("s")
        wid = c * NS + s
        pltpu.sync_copy(zagg_hbm, acc.at[pl.ds(s * RPS, RPS)])
        plsc.subcore_barrier()

        def run_half(h):
            pltpu.sync_copy(src_hbm.at[wid, pl.ds(h * HCH, HCH)], src_v)
            pltpu.sync_copy(dst_hbm.at[wid, pl.ds(h * HCH, HCH)], dst_v)
            pltpu.async_copy(hs_hbm.at[src_v.at[0]], rows0, sem0)

            def step(t, carry):
                j = 2 * t
                pltpu.make_async_copy(hs_hbm.at[src_v.at[j]], rows0,
                                      sem0).wait()
                pltpu.async_copy(hs_hbm.at[src_v.at[j + 1]], rows1, sem1)
                pltpu.sync_copy(rows0, acc.at[dst_v.at[j]], add=True)
                pltpu.make_async_copy(hs_hbm.at[src_v.at[j + 1]], rows1,
                                      sem1).wait()
                pltpu.async_copy(hs_hbm.at[src_v.at[j + 2]], rows0, sem0)
                pltpu.sync_copy(rows1, acc.at[dst_v.at[j + 1]], add=True)
                return carry

            lax.fori_loop(0, HCH // 2 - 1, step, 0)
            jt = HCH - 2
            pltpu.make_async_copy(hs_hbm.at[src_v.at[jt]], rows0, sem0).wait()
            pltpu.async_copy(hs_hbm.at[src_v.at[HCH - 1]], rows1, sem1)
            pltpu.sync_copy(rows0, acc.at[dst_v.at[jt]], add=True)
            pltpu.make_async_copy(hs_hbm.at[src_v.at[HCH - 1]], rows1,
                                  sem1).wait()
            pltpu.sync_copy(rows1, acc.at[dst_v.at[HCH - 1]], add=True)

        run_half(0)
        run_half(1)

        plsc.subcore_barrier()
        pltpu.sync_copy(acc.at[pl.ds(s * RPS, RPS)],
                        out_hbm.at[c, pl.ds(s * RPS, RPS)])

    return sc_deg, sc_agg


def _dis_from_deg(deg):
    d = deg[0, :N, :1] + deg[1, :N, :1] + 1.0
    return lax.rsqrt(jnp.maximum(d, 1.0))


def _tc_dense1(x_ref, w_ref, deg_ref, out_ref):
    dis = _dis_from_deg(deg_ref[...])
    h = jnp.dot(x_ref[...], w_ref[...], preferred_element_type=jnp.float32)
    out_ref[...] = h * dis


def _bn_relu(agg, hs, dis, b, gamma, beta):
    h = dis * (agg[0, :N] + agg[1, :N] + hs) + b
    mean = jnp.mean(h, axis=0, keepdims=True)
    var = jnp.mean((h - mean) ** 2, axis=0, keepdims=True)
    h = (h - mean) * lax.rsqrt(var + EPS) * gamma + beta
    return jnp.maximum(h, 0.0)


def _tc_dense2(agg_ref, hs_ref, deg_ref, b_ref, g_ref, be_ref, w_ref, out_ref):
    dis = _dis_from_deg(deg_ref[...])
    h = _bn_relu(agg_ref[...], hs_ref[...], dis, b_ref[...], g_ref[...],
                 be_ref[...])
    h2 = jnp.dot(h, w_ref[...], preferred_element_type=jnp.float32)
    out_ref[...] = h2 * dis


def _tc_dense3(agg_ref, hs_ref, deg_ref, b_ref, g_ref, be_ref, batch_ref,
               out_ref):
    dis = _dis_from_deg(deg_ref[...])
    h = _bn_relu(agg_ref[...], hs_ref[...], dis, b_ref[...], g_ref[...],
                 be_ref[...])
    seg = lax.broadcasted_iota(jnp.int32, (N, G), 1)
    onehot = (batch_ref[...] == seg).astype(jnp.float32)
    sums = lax.dot_general(onehot, h, (((0,), (0,)), ((), ())),
                           preferred_element_type=jnp.float32)
    counts = jnp.sum(onehot, axis=0)[:, None]
    out_ref[...] = sums / jnp.maximum(counts, 1.0)


def kernel(x, edge_index, batch, W1, b1, gamma1, beta1, W2, b2, gamma2, beta2):
    pad_idx = jnp.arange(E_PAD - E, dtype=jnp.int32)
    pad_src = pad_idx % N
    pad_dst = N + (pad_idx % (NP - N))
    src = jnp.concatenate([edge_index[0], pad_src]).reshape(NW, CH, K)
    dst = jnp.concatenate([edge_index[1], pad_dst]).reshape(NW, CH, K)
    batch2 = batch.reshape(N, 1)
    b1r, g1r, be1r = b1.reshape(1, D), gamma1.reshape(1, D), beta1.reshape(1, D)
    b2r, g2r, be2r = b2.reshape(1, D), gamma2.reshape(1, D), beta2.reshape(1, D)

    ones_deg = jnp.ones((K, D), jnp.float32)
    zagg = jnp.zeros((RPS, D), jnp.float32)

    sc_deg, sc_agg = _sc_kernels()
    deg = sc_deg(dst, ones_deg, zagg)

    hs1 = pl.pallas_call(
        _tc_dense1,
        out_shape=jax.ShapeDtypeStruct((N, D), jnp.float32),
    )(x, W1, deg)

    agg1 = sc_agg(hs1, src, dst, zagg)

    hs2 = pl.pallas_call(
        _tc_dense2,
        out_shape=jax.ShapeDtypeStruct((N, D), jnp.float32),
    )(agg1, hs1, deg, b1r, g1r, be1r, W2)

    agg2 = sc_agg(hs2, src, dst, zagg)

    out = pl.pallas_call(
        _tc_dense3,
        out_shape=jax.ShapeDtypeStruct((G, D), jnp.float32),
    )(agg2, hs2, deg, b2r, g2r, be2r, batch2)

    return out


# speedup vs baseline: 3.4557x; 1.2131x over previous
import functools

import jax
import jax.numpy as jnp
from jax import lax
from jax.experimental import pallas as pl
from jax.experimental.pallas import tpu as pltpu
from jax.experimental.pallas import tpu_sc as plsc

N = 10000
E = 320000
D = 128
G = 64
EPS = 1e-5

NC = 2
NS = 16
NW = NC * NS
EPW = E // NW
K = 128
CH = 80
HCH = CH // 2
E_PAD = NW * CH * K
NP = 10240
RPS = NP // NS


@functools.cache
def _sc_kernels():
    mesh = plsc.VectorSubcoreMesh(core_axis_name="c", subcore_axis_name="s",
                                  num_cores=NC, num_subcores=NS)

    @functools.partial(
        pl.kernel,
        out_type=(jax.ShapeDtypeStruct((NP,), jnp.float32),
                  jax.ShapeDtypeStruct((NP,), jnp.float32)),
        mesh=mesh,
        scratch_types=[
            pltpu.VMEM((CH, K), jnp.int32),
            pltpu.VMEM((K,), jnp.float32),
            pltpu.VMEM_SHARED((NP,), jnp.float32),
        ],
    )
    def sc_deg(dst_hbm, ones_hbm, zdeg_hbm, out0_hbm, out1_hbm,
               dst_v, ones_v, acc):
        c = lax.axis_index("c")
        s = lax.axis_index("s")
        wid = c * NS + s
        pltpu.sync_copy(dst_hbm.at[wid], dst_v)
        pltpu.sync_copy(ones_hbm, ones_v)
        pltpu.sync_copy(zdeg_hbm, acc.at[pl.ds(s * RPS, RPS)])
        plsc.subcore_barrier()

        def step(j, carry):
            pltpu.sync_copy(ones_v, acc.at[dst_v.at[j]], add=True)
            return carry

        lax.fori_loop(0, CH, step, 0)
        plsc.subcore_barrier()

        @pl.when(c == 0)
        def _():
            pltpu.sync_copy(acc.at[pl.ds(s * RPS, RPS)],
                            out0_hbm.at[pl.ds(s * RPS, RPS)])

        @pl.when(c == 1)
        def _():
            pltpu.sync_copy(acc.at[pl.ds(s * RPS, RPS)],
                            out1_hbm.at[pl.ds(s * RPS, RPS)])

    @functools.partial(
        pl.kernel,
        out_type=jax.ShapeDtypeStruct((NC, NP, D), jnp.float32),
        mesh=mesh,
        scratch_types=[
            pltpu.VMEM((HCH, K), jnp.int32),
            pltpu.VMEM((HCH, K), jnp.int32),
            pltpu.VMEM((K, D), jnp.float32),
            pltpu.VMEM((K, D), jnp.float32),
            pltpu.SemaphoreType.DMA,
            pltpu.SemaphoreType.DMA,
            pltpu.VMEM_SHARED((NP, D), jnp.float32),
        ],
    )
    def sc_agg(hs_hbm, src_hbm, dst_hbm, zagg_hbm, out_hbm,
               src_v, dst_v, rows0, rows1, sem0, sem1, acc):
        c = lax.axis_index("c")
        s = lax.axis_index("s")
        wid = c * NS + s
        pltpu.sync_copy(zagg_hbm, acc.at[pl.ds(s * RPS, RPS)])
        plsc.subcore_barrier()

        def run_half(h):
            pltpu.sync_copy(src_hbm.at[wid, pl.ds(h * HCH, HCH)], src_v)
            pltpu.sync_copy(dst_hbm.at[wid, pl.ds(h * HCH, HCH)], dst_v)
            pltpu.async_copy(hs_hbm.at[src_v.at[0]], rows0, sem0)

            def step(t, carry):
                j = 2 * t
                pltpu.make_async_copy(hs_hbm.at[src_v.at[j]], rows0,
                                      sem0).wait()
                pltpu.async_copy(hs_hbm.at[src_v.at[j + 1]], rows1, sem1)
                pltpu.sync_copy(rows0, acc.at[dst_v.at[j]], add=True)
                pltpu.make_async_copy(hs_hbm.at[src_v.at[j + 1]], rows1,
                                      sem1).wait()
                pltpu.async_copy(hs_hbm.at[src_v.at[j + 2]], rows0, sem0)
                pltpu.sync_copy(rows1, acc.at[dst_v.at[j + 1]], add=True)
                return carry

            lax.fori_loop(0, HCH // 2 - 1, step, 0)
            jt = HCH - 2
            pltpu.make_async_copy(hs_hbm.at[src_v.at[jt]], rows0, sem0).wait()
            pltpu.async_copy(hs_hbm.at[src_v.at[HCH - 1]], rows1, sem1)
            pltpu.sync_copy(rows0, acc.at[dst_v.at[jt]], add=True)
            pltpu.make_async_copy(hs_hbm.at[src_v.at[HCH - 1]], rows1,
                                  sem1).wait()
            pltpu.sync_copy(rows1, acc.at[dst_v.at[HCH - 1]], add=True)

        run_half(0)
        run_half(1)

        plsc.subcore_barrier()
        pltpu.sync_copy(acc.at[pl.ds(s * RPS, RPS)],
                        out_hbm.at[c, pl.ds(s * RPS, RPS)])

    return sc_deg, sc_agg


def _dis_from_deg(deg0, deg1):
    d = deg0[:N] + deg1[:N] + 1.0
    return lax.rsqrt(jnp.maximum(d, 1.0))[:, None]


def _tc_dense1(x_ref, w_ref, deg0_ref, deg1_ref, out_ref):
    dis = _dis_from_deg(deg0_ref[...], deg1_ref[...])
    h = jnp.dot(x_ref[...], w_ref[...], preferred_element_type=jnp.float32)
    out_ref[...] = h * dis


def _bn_relu(agg, hs, dis, b, gamma, beta):
    h = dis * (agg[0, :N] + agg[1, :N] + hs) + b
    mean = jnp.mean(h, axis=0, keepdims=True)
    var = jnp.mean((h - mean) ** 2, axis=0, keepdims=True)
    h = (h - mean) * lax.rsqrt(var + EPS) * gamma + beta
    return jnp.maximum(h, 0.0)


def _tc_dense2(agg_ref, hs_ref, deg0_ref, deg1_ref, b_ref, g_ref, be_ref,
               w_ref, out_ref):
    dis = _dis_from_deg(deg0_ref[...], deg1_ref[...])
    h = _bn_relu(agg_ref[...], hs_ref[...], dis, b_ref[...], g_ref[...],
                 be_ref[...])
    h2 = jnp.dot(h, w_ref[...], preferred_element_type=jnp.float32)
    out_ref[...] = h2 * dis


def _tc_dense3(agg_ref, hs_ref, deg0_ref, deg1_ref, b_ref, g_ref, be_ref,
               batch_ref, out_ref):
    dis = _dis_from_deg(deg0_ref[...], deg1_ref[...])
    h = _bn_relu(agg_ref[...], hs_ref[...], dis, b_ref[...], g_ref[...],
                 be_ref[...])
    seg = lax.broadcasted_iota(jnp.int32, (N, G), 1)
    onehot = (batch_ref[...] == seg).astype(jnp.float32)
    sums = lax.dot_general(onehot, h, (((0,), (0,)), ((), ())),
                           preferred_element_type=jnp.float32)
    counts = jnp.sum(onehot, axis=0)[:, None]
    out_ref[...] = sums / jnp.maximum(counts, 1.0)


def kernel(x, edge_index, batch, W1, b1, gamma1, beta1, W2, b2, gamma2, beta2):
    pad_idx = jnp.arange(E_PAD - E, dtype=jnp.int32)
    pad_src = pad_idx % N
    pad_dst = N + (pad_idx % (NP - N))
    src = jnp.concatenate([edge_index[0], pad_src]).reshape(NW, CH, K)
    dst = jnp.concatenate([edge_index[1], pad_dst]).reshape(NW, CH, K)
    batch2 = batch.reshape(N, 1)
    b1r, g1r, be1r = b1.reshape(1, D), gamma1.reshape(1, D), beta1.reshape(1, D)
    b2r, g2r, be2r = b2.reshape(1, D), gamma2.reshape(1, D), beta2.reshape(1, D)

    ones_deg = jnp.ones((K,), jnp.float32)
    zdeg = jnp.zeros((RPS,), jnp.float32)
    zagg = jnp.zeros((RPS, D), jnp.float32)

    sc_deg, sc_agg = _sc_kernels()
    deg0, deg1 = sc_deg(dst, ones_deg, zdeg)

    hs1 = pl.pallas_call(
        _tc_dense1,
        out_shape=jax.ShapeDtypeStruct((N, D), jnp.float32),
    )(x, W1, deg0, deg1)

    agg1 = sc_agg(hs1, src, dst, zagg)

    hs2 = pl.pallas_call(
        _tc_dense2,
        out_shape=jax.ShapeDtypeStruct((N, D), jnp.float32),
    )(agg1, hs1, deg0, deg1, b1r, g1r, be1r, W2)

    agg2 = sc_agg(hs2, src, dst, zagg)

    out = pl.pallas_call(
        _tc_dense3,
        out_shape=jax.ShapeDtypeStruct((G, D), jnp.float32),
    )(agg2, hs2, deg0, deg1, b2r, g2r, be2r, batch2)

    return out


# speedup vs baseline: 3.5351x; 1.0230x over previous
import functools

import jax
import jax.numpy as jnp
import numpy as np
from jax import lax
from jax.experimental import pallas as pl
from jax.experimental.pallas import tpu as pltpu
from jax.experimental.pallas import tpu_sc as plsc

N = 10000
E = 320000
D = 128
G = 64
EPS = 1e-5

NC = 2
NS = 16
NW = NC * NS
EPW = E // NW
K = 128
CH = 80
HCH = CH // 2
E_PAD = NW * CH * K
NP = 10240
RPS = NP // NS


@functools.cache
def _sc_kernels():
    mesh = plsc.VectorSubcoreMesh(core_axis_name="c", subcore_axis_name="s",
                                  num_cores=NC, num_subcores=NS)

    @functools.partial(
        pl.kernel,
        out_type=(jax.ShapeDtypeStruct((NP,), jnp.float32),
                  jax.ShapeDtypeStruct((NP,), jnp.float32)),
        mesh=mesh,
        scratch_types=[
            pltpu.VMEM((CH, K), jnp.int32),
            pltpu.VMEM((K,), jnp.float32),
            pltpu.VMEM_SHARED((NP,), jnp.float32),
        ],
    )
    def sc_deg(dst_hbm, dstt_hbm, ones_hbm, zdeg_hbm, out0_hbm, out1_hbm,
               dst_v, ones_v, acc):
        c = lax.axis_index("c")
        s = lax.axis_index("s")
        wid = c * NS + s

        @pl.when(wid < NW - 1)
        def _():
            pltpu.sync_copy(dst_hbm.at[wid], dst_v)

        @pl.when(wid == NW - 1)
        def _():
            pltpu.sync_copy(dstt_hbm, dst_v)

        pltpu.sync_copy(ones_hbm, ones_v)
        pltpu.sync_copy(zdeg_hbm, acc.at[pl.ds(s * RPS, RPS)])
        plsc.subcore_barrier()

        def step(j, carry):
            pltpu.sync_copy(ones_v, acc.at[dst_v.at[j]], add=True)
            return carry

        lax.fori_loop(0, CH, step, 0)
        plsc.subcore_barrier()

        @pl.when(c == 0)
        def _():
            pltpu.sync_copy(acc.at[pl.ds(s * RPS, RPS)],
                            out0_hbm.at[pl.ds(s * RPS, RPS)])

        @pl.when(c == 1)
        def _():
            pltpu.sync_copy(acc.at[pl.ds(s * RPS, RPS)],
                            out1_hbm.at[pl.ds(s * RPS, RPS)])

    @functools.partial(
        pl.kernel,
        out_type=jax.ShapeDtypeStruct((NC, NP, D), jnp.float32),
        mesh=mesh,
        scratch_types=[
            pltpu.VMEM((HCH, K), jnp.int32),
            pltpu.VMEM((HCH, K), jnp.int32),
            pltpu.VMEM((K, D), jnp.float32),
            pltpu.VMEM((K, D), jnp.float32),
            pltpu.SemaphoreType.DMA,
            pltpu.SemaphoreType.DMA,
            pltpu.VMEM_SHARED((NP, D), jnp.float32),
        ],
    )
    def sc_agg(hs_hbm, src_hbm, srct_hbm, dst_hbm, dstt_hbm, zagg_hbm,
               out_hbm, src_v, dst_v, rows0, rows1, sem0, sem1, acc):
        c = lax.axis_index("c")
        s = lax.axis_index("s")
        wid = c * NS + s
        pltpu.sync_copy(zagg_hbm, acc.at[pl.ds(s * RPS, RPS)])
        plsc.subcore_barrier()

        def run_half(h):
            @pl.when(wid < NW - 1)
            def _():
                pltpu.sync_copy(src_hbm.at[wid, pl.ds(h * HCH, HCH)], src_v)
                pltpu.sync_copy(dst_hbm.at[wid, pl.ds(h * HCH, HCH)], dst_v)

            @pl.when(wid == NW - 1)
            def _():
                pltpu.sync_copy(srct_hbm.at[pl.ds(h * HCH, HCH)], src_v)
                pltpu.sync_copy(dstt_hbm.at[pl.ds(h * HCH, HCH)], dst_v)
            pltpu.async_copy(hs_hbm.at[src_v.at[0]], rows0, sem0)

            def step(t, carry):
                j = 2 * t
                pltpu.make_async_copy(hs_hbm.at[src_v.at[j]], rows0,
                                      sem0).wait()
                pltpu.async_copy(hs_hbm.at[src_v.at[j + 1]], rows1, sem1)
                pltpu.sync_copy(rows0, acc.at[dst_v.at[j]], add=True)
                pltpu.make_async_copy(hs_hbm.at[src_v.at[j + 1]], rows1,
                                      sem1).wait()
                pltpu.async_copy(hs_hbm.at[src_v.at[j + 2]], rows0, sem0)
                pltpu.sync_copy(rows1, acc.at[dst_v.at[j + 1]], add=True)
                return carry

            lax.fori_loop(0, HCH // 2 - 1, step, 0)
            jt = HCH - 2
            pltpu.make_async_copy(hs_hbm.at[src_v.at[jt]], rows0, sem0).wait()
            pltpu.async_copy(hs_hbm.at[src_v.at[HCH - 1]], rows1, sem1)
            pltpu.sync_copy(rows0, acc.at[dst_v.at[jt]], add=True)
            pltpu.make_async_copy(hs_hbm.at[src_v.at[HCH - 1]], rows1,
                                  sem1).wait()
            pltpu.sync_copy(rows1, acc.at[dst_v.at[HCH - 1]], add=True)

        run_half(0)
        run_half(1)

        plsc.subcore_barrier()
        pltpu.sync_copy(acc.at[pl.ds(s * RPS, RPS)],
                        out_hbm.at[c, pl.ds(s * RPS, RPS)])

    return sc_deg, sc_agg


def _dis_from_deg(deg0, deg1):
    d = deg0[:N] + deg1[:N] + 1.0
    return lax.rsqrt(jnp.maximum(d, 1.0))[:, None]


def _tc_dense1(x_ref, w_ref, deg0_ref, deg1_ref, out_ref):
    dis = _dis_from_deg(deg0_ref[...], deg1_ref[...])
    h = jnp.dot(x_ref[...], w_ref[...], preferred_element_type=jnp.float32)
    out_ref[...] = h * dis


def _bn_relu(agg, hs, dis, b, gamma, beta):
    h = dis * (agg[0, :N] + agg[1, :N] + hs) + b
    mean = jnp.mean(h, axis=0, keepdims=True)
    var = jnp.mean((h - mean) ** 2, axis=0, keepdims=True)
    h = (h - mean) * lax.rsqrt(var + EPS) * gamma + beta
    return jnp.maximum(h, 0.0)


def _tc_dense2(agg_ref, hs_ref, deg0_ref, deg1_ref, b_ref, g_ref, be_ref,
               w_ref, out_ref):
    dis = _dis_from_deg(deg0_ref[...], deg1_ref[...])
    h = _bn_relu(agg_ref[...], hs_ref[...], dis, b_ref[...], g_ref[...],
                 be_ref[...])
    h2 = jnp.dot(h, w_ref[...], preferred_element_type=jnp.float32)
    out_ref[...] = h2 * dis


def _tc_dense3(agg_ref, hs_ref, deg0_ref, deg1_ref, b_ref, g_ref, be_ref,
               batch_ref, out_ref):
    dis = _dis_from_deg(deg0_ref[...], deg1_ref[...])
    h = _bn_relu(agg_ref[...], hs_ref[...], dis, b_ref[...], g_ref[...],
                 be_ref[...])
    seg = lax.broadcasted_iota(jnp.int32, (N, G), 1)
    onehot = (batch_ref[...] == seg).astype(jnp.float32)
    sums = lax.dot_general(onehot, h, (((0,), (0,)), ((), ())),
                           preferred_element_type=jnp.float32)
    counts = jnp.sum(onehot, axis=0)[:, None]
    out_ref[...] = sums / jnp.maximum(counts, 1.0)


_E_MAIN = (NW - 1) * CH * K
_PAD_IDX = np.arange(E_PAD - E, dtype=np.int32)
_PAD_SRC = jnp.asarray(_PAD_IDX % N)
_PAD_DST = jnp.asarray(N + (_PAD_IDX % (NP - N)))


def kernel(x, edge_index, batch, W1, b1, gamma1, beta1, W2, b2, gamma2, beta2):
    src = edge_index[0, :_E_MAIN].reshape(NW - 1, CH, K)
    dst = edge_index[1, :_E_MAIN].reshape(NW - 1, CH, K)
    src_t = jnp.concatenate([edge_index[0, _E_MAIN:], _PAD_SRC]).reshape(CH, K)
    dst_t = jnp.concatenate([edge_index[1, _E_MAIN:], _PAD_DST]).reshape(CH, K)
    batch2 = batch.reshape(N, 1)
    b1r, g1r, be1r = b1.reshape(1, D), gamma1.reshape(1, D), beta1.reshape(1, D)
    b2r, g2r, be2r = b2.reshape(1, D), gamma2.reshape(1, D), beta2.reshape(1, D)

    ones_deg = jnp.ones((K,), jnp.float32)
    zdeg = jnp.zeros((RPS,), jnp.float32)
    zagg = jnp.zeros((RPS, D), jnp.float32)

    sc_deg, sc_agg = _sc_kernels()
    deg0, deg1 = sc_deg(dst, dst_t, ones_deg, zdeg)

    hs1 = pl.pallas_call(
        _tc_dense1,
        out_shape=jax.ShapeDtypeStruct((N, D), jnp.float32),
    )(x, W1, deg0, deg1)

    agg1 = sc_agg(hs1, src, src_t, dst, dst_t, zagg)

    hs2 = pl.pallas_call(
        _tc_dense2,
        out_shape=jax.ShapeDtypeStruct((N, D), jnp.float32),
    )(agg1, hs1, deg0, deg1, b1r, g1r, be1r, W2)

    agg2 = sc_agg(hs2, src, src_t, dst, dst_t, zagg)

    out = pl.pallas_call(
        _tc_dense3,
        out_shape=jax.ShapeDtypeStruct((G, D), jnp.float32),
    )(agg2, hs2, deg0, deg1, b2r, g2r, be2r, batch2)

    return out


# speedup vs baseline: 4.0360x; 1.1417x over previous
import functools

import jax
import jax.numpy as jnp
import numpy as np
from jax import lax
from jax.experimental import pallas as pl
from jax.experimental.pallas import tpu as pltpu
from jax.experimental.pallas import tpu_sc as plsc

N = 10000
E = 320000
D = 128
G = 64
EPS = 1e-5

NC = 2
NS = 16
NW = NC * NS
EPW = E // NW
K = 128
CH = 80
HCH = CH // 2
E_PAD = NW * CH * K
NP = 10240
RPS = NP // NS


@functools.cache
def _sc_kernels():
    mesh = plsc.VectorSubcoreMesh(core_axis_name="c", subcore_axis_name="s",
                                  num_cores=NC, num_subcores=NS)

    @functools.partial(
        pl.kernel,
        out_type=(jax.ShapeDtypeStruct((NP,), jnp.float32),
                  jax.ShapeDtypeStruct((NP,), jnp.float32)),
        mesh=mesh,
        scratch_types=[
            pltpu.VMEM((CH, K), jnp.int32),
            pltpu.VMEM((K,), jnp.float32),
            pltpu.VMEM_SHARED((NP,), jnp.float32),
        ],
    )
    def sc_deg(dst_hbm, dstt_hbm, ones_hbm, zdeg_hbm, out0_hbm, out1_hbm,
               dst_v, ones_v, acc):
        c = lax.axis_index("c")
        s = lax.axis_index("s")
        wid = c * NS + s

        @pl.when(wid < NW - 1)
        def _():
            pltpu.sync_copy(dst_hbm.at[wid], dst_v)

        @pl.when(wid == NW - 1)
        def _():
            pltpu.sync_copy(dstt_hbm, dst_v)

        pltpu.sync_copy(ones_hbm, ones_v)
        pltpu.sync_copy(zdeg_hbm, acc.at[pl.ds(s * RPS, RPS)])
        plsc.subcore_barrier()

        def step(j, carry):
            pltpu.sync_copy(ones_v, acc.at[dst_v.at[j]], add=True)
            return carry

        lax.fori_loop(0, CH, step, 0)
        plsc.subcore_barrier()

        @pl.when(c == 0)
        def _():
            pltpu.sync_copy(acc.at[pl.ds(s * RPS, RPS)],
                            out0_hbm.at[pl.ds(s * RPS, RPS)])

        @pl.when(c == 1)
        def _():
            pltpu.sync_copy(acc.at[pl.ds(s * RPS, RPS)],
                            out1_hbm.at[pl.ds(s * RPS, RPS)])

    @functools.partial(
        pl.kernel,
        out_type=jax.ShapeDtypeStruct((NC, NP, D), jnp.float32),
        mesh=mesh,
        scratch_types=[
            pltpu.VMEM((HCH, K), jnp.int32),
            pltpu.VMEM((HCH, K), jnp.int32),
            pltpu.VMEM((K, D), jnp.float32),
            pltpu.VMEM((K, D), jnp.float32),
            pltpu.SemaphoreType.DMA,
            pltpu.SemaphoreType.DMA,
            pltpu.VMEM_SHARED((NP, D), jnp.float32),
        ],
    )
    def sc_agg(hs_hbm, src_hbm, srct_hbm, dst_hbm, dstt_hbm, zagg_hbm,
               out_hbm, src_v, dst_v, rows0, rows1, sem0, sem1, acc):
        c = lax.axis_index("c")
        s = lax.axis_index("s")
        wid = c * NS + s
        pltpu.sync_copy(zagg_hbm, acc.at[pl.ds(s * RPS, RPS)])
        plsc.subcore_barrier()

        def run_half(h):
            @pl.when(wid < NW - 1)
            def _():
                pltpu.sync_copy(src_hbm.at[wid, pl.ds(h * HCH, HCH)], src_v)
                pltpu.sync_copy(dst_hbm.at[wid, pl.ds(h * HCH, HCH)], dst_v)

            @pl.when(wid == NW - 1)
            def _():
                pltpu.sync_copy(srct_hbm.at[pl.ds(h * HCH, HCH)], src_v)
                pltpu.sync_copy(dstt_hbm.at[pl.ds(h * HCH, HCH)], dst_v)
            pltpu.async_copy(hs_hbm.at[src_v.at[0]], rows0, sem0)

            def step(t, carry):
                j = 2 * t
                pltpu.async_copy(hs_hbm.at[src_v.at[j + 1]], rows1, sem1)
                pltpu.make_async_copy(hs_hbm.at[src_v.at[j]], rows0,
                                      sem0).wait()
                pltpu.sync_copy(rows0, acc.at[dst_v.at[j]], add=True)
                pltpu.async_copy(hs_hbm.at[src_v.at[j + 2]], rows0, sem0)
                pltpu.make_async_copy(hs_hbm.at[src_v.at[j + 1]], rows1,
                                      sem1).wait()
                pltpu.sync_copy(rows1, acc.at[dst_v.at[j + 1]], add=True)
                return carry

            lax.fori_loop(0, HCH // 2 - 1, step, 0)
            jt = HCH - 2
            pltpu.async_copy(hs_hbm.at[src_v.at[HCH - 1]], rows1, sem1)
            pltpu.make_async_copy(hs_hbm.at[src_v.at[jt]], rows0, sem0).wait()
            pltpu.sync_copy(rows0, acc.at[dst_v.at[jt]], add=True)
            pltpu.make_async_copy(hs_hbm.at[src_v.at[HCH - 1]], rows1,
                                  sem1).wait()
            pltpu.sync_copy(rows1, acc.at[dst_v.at[HCH - 1]], add=True)

        run_half(0)
        run_half(1)

        plsc.subcore_barrier()
        pltpu.sync_copy(acc.at[pl.ds(s * RPS, RPS)],
                        out_hbm.at[c, pl.ds(s * RPS, RPS)])

    return sc_deg, sc_agg


def _dis_from_deg(deg0, deg1):
    d = deg0[:N] + deg1[:N] + 1.0
    return lax.rsqrt(jnp.maximum(d, 1.0))[:, None]


def _tc_dense1(x_ref, w_ref, deg0_ref, deg1_ref, out_ref):
    dis = _dis_from_deg(deg0_ref[...], deg1_ref[...])
    h = jnp.dot(x_ref[...], w_ref[...], preferred_element_type=jnp.float32)
    out_ref[...] = h * dis


def _bn_relu(agg, hs, dis, b, gamma, beta):
    h = dis * (agg[0, :N] + agg[1, :N] + hs) + b
    mean = jnp.mean(h, axis=0, keepdims=True)
    var = jnp.mean((h - mean) ** 2, axis=0, keepdims=True)
    h = (h - mean) * lax.rsqrt(var + EPS) * gamma + beta
    return jnp.maximum(h, 0.0)


def _tc_dense2(agg_ref, hs_ref, deg0_ref, deg1_ref, b_ref, g_ref, be_ref,
               w_ref, out_ref):
    dis = _dis_from_deg(deg0_ref[...], deg1_ref[...])
    h = _bn_relu(agg_ref[...], hs_ref[...], dis, b_ref[...], g_ref[...],
                 be_ref[...])
    h2 = jnp.dot(h, w_ref[...], preferred_element_type=jnp.float32)
    out_ref[...] = h2 * dis


def _tc_dense3(agg_ref, hs_ref, deg0_ref, deg1_ref, b_ref, g_ref, be_ref,
               batch_ref, out_ref):
    dis = _dis_from_deg(deg0_ref[...], deg1_ref[...])
    h = _bn_relu(agg_ref[...], hs_ref[...], dis, b_ref[...], g_ref[...],
                 be_ref[...])
    seg = lax.broadcasted_iota(jnp.int32, (N, G), 1)
    onehot = (batch_ref[...] == seg).astype(jnp.float32)
    sums = lax.dot_general(onehot, h, (((0,), (0,)), ((), ())),
                           preferred_element_type=jnp.float32)
    counts = jnp.sum(onehot, axis=0)[:, None]
    out_ref[...] = sums / jnp.maximum(counts, 1.0)


_E_MAIN = (NW - 1) * CH * K
_PAD_IDX = np.arange(E_PAD - E, dtype=np.int32)
_PAD_SRC = jnp.asarray(_PAD_IDX % N)
_PAD_DST = jnp.asarray(N + (_PAD_IDX % (NP - N)))


def kernel(x, edge_index, batch, W1, b1, gamma1, beta1, W2, b2, gamma2, beta2):
    src = edge_index[0, :_E_MAIN].reshape(NW - 1, CH, K)
    dst = edge_index[1, :_E_MAIN].reshape(NW - 1, CH, K)
    src_t = jnp.concatenate([edge_index[0, _E_MAIN:], _PAD_SRC]).reshape(CH, K)
    dst_t = jnp.concatenate([edge_index[1, _E_MAIN:], _PAD_DST]).reshape(CH, K)
    batch2 = batch.reshape(N, 1)
    b1r, g1r, be1r = b1.reshape(1, D), gamma1.reshape(1, D), beta1.reshape(1, D)
    b2r, g2r, be2r = b2.reshape(1, D), gamma2.reshape(1, D), beta2.reshape(1, D)

    ones_deg = jnp.ones((K,), jnp.float32)
    zdeg = jnp.zeros((RPS,), jnp.float32)
    zagg = jnp.zeros((RPS, D), jnp.float32)

    sc_deg, sc_agg = _sc_kernels()
    deg0, deg1 = sc_deg(dst, dst_t, ones_deg, zdeg)

    hs1 = pl.pallas_call(
        _tc_dense1,
        out_shape=jax.ShapeDtypeStruct((N, D), jnp.float32),
    )(x, W1, deg0, deg1)

    agg1 = sc_agg(hs1, src, src_t, dst, dst_t, zagg)

    hs2 = pl.pallas_call(
        _tc_dense2,
        out_shape=jax.ShapeDtypeStruct((N, D), jnp.float32),
    )(agg1, hs1, deg0, deg1, b1r, g1r, be1r, W2)

    agg2 = sc_agg(hs2, src, src_t, dst, dst_t, zagg)

    out = pl.pallas_call(
        _tc_dense3,
        out_shape=jax.ShapeDtypeStruct((G, D), jnp.float32),
    )(agg2, hs2, deg0, deg1, b2r, g2r, be2r, batch2)

    return out
